# Initial kernel scaffold; baseline (speedup 1.0000x reference)
import jax
import jax.numpy as jnp
from jax.experimental import pallas as pl


def kernel(features, tables, W, b, gamma, beta):
    raise NotImplementedError("write your pallas kernel here")


# speedup vs baseline: 7.9922x; 7.9922x over previous
import functools

import jax
import jax.numpy as jnp
from jax import lax
from jax.experimental import pallas as pl
from jax.experimental.pallas import tpu as pltpu
from jax.experimental.pallas import tpu_sc as plsc

_EPS = 1e-5

_NC = 2
_NS = 16
_NW = _NC * _NS


@functools.partial(jax.jit, static_argnames=("chunk",))
def _sc_gather(gidx, tab_flat, chunk=3328):
    n, = gidx.shape
    _, d = tab_flat.shape
    n_per_w = n // _NW
    assert n_per_w * _NW == n and n_per_w % chunk == 0
    n_ch = n_per_w // chunk

    mesh = plsc.VectorSubcoreMesh(
        core_axis_name="c", subcore_axis_name="s",
        num_cores=_NC, num_subcores=_NS)

    @functools.partial(
        pl.kernel,
        out_type=jax.ShapeDtypeStruct((n, d), jnp.float32),
        mesh=mesh,
        scratch_types=[
            pltpu.VMEM((chunk,), jnp.int32),
            pltpu.VMEM((chunk, d), jnp.float32),
            pltpu.SemaphoreType.DMA,
        ],
        compiler_params=pltpu.CompilerParams(use_tc_tiling_on_sc=False),
    )
    def gather(idx_hbm, tab_hbm, out_hbm, idx_v, rows_v, sem):
        wid = lax.axis_index("s") * _NC + lax.axis_index("c")
        base = wid * n_per_w
        for k in range(n_ch):
            off = base + k * chunk
            pltpu.sync_copy(idx_hbm.at[pl.ds(off, chunk)], idx_v)
            pltpu.async_copy(tab_hbm.at[idx_v], rows_v, sem).wait()
            pltpu.sync_copy(rows_v, out_hbm.at[pl.ds(off, chunk)])

    return gather(gidx, tab_flat)


def _mm_body(x_ref, w_ref, b_ref, h_ref):
    h_ref[...] = lax.dot_general(
        x_ref[...], w_ref[...], (((1,), (1,)), ((), ())),
        preferred_element_type=jnp.float32) + b_ref[...]


def _bn_body(h_ref, g_ref, bt_ref, o_ref):
    h = h_ref[...]
    mu = jnp.mean(h, axis=0, keepdims=True)
    var = jnp.mean((h - mu) ** 2, axis=0, keepdims=True)
    o_ref[...] = jnp.maximum(
        (h - mu) * lax.rsqrt(var + _EPS) * g_ref[...] + bt_ref[...], 0.0)


def kernel(features, tables, W, b, gamma, beta):
    bsz, f_num = features.shape
    _, vocab, d = tables.shape
    n = bsz * f_num

    gidx = (features.astype(jnp.int32)
            + (jnp.arange(f_num, dtype=jnp.int32) * vocab)[None, :]).reshape(n)
    tab_flat = tables.reshape(f_num * vocab, d)

    x = _sc_gather(gidx, tab_flat).reshape(bsz, f_num * d)

    blk = 512
    h = pl.pallas_call(
        _mm_body,
        grid=(bsz // blk,),
        in_specs=[
            pl.BlockSpec((blk, f_num * d), lambda i: (i, 0)),
            pl.BlockSpec((d, f_num * d), lambda i: (0, 0)),
            pl.BlockSpec((1, d), lambda i: (0, 0)),
        ],
        out_specs=pl.BlockSpec((blk, d), lambda i: (i, 0)),
        out_shape=jax.ShapeDtypeStruct((bsz, d), jnp.float32),
    )(x, W, b.reshape(1, d))

    out = pl.pallas_call(
        _bn_body,
        out_shape=jax.ShapeDtypeStruct((bsz, d), jnp.float32),
    )(h, gamma.reshape(1, d), beta.reshape(1, d))
    return out


# speedup vs baseline: 12.9052x; 1.6147x over previous
import functools

import jax
import jax.numpy as jnp
from jax import lax
from jax.experimental import pallas as pl
from jax.experimental.pallas import tpu as pltpu
from jax.experimental.pallas import tpu_sc as plsc

_EPS = 1e-5

_NC = 2
_NS = 16
_NW = _NC * _NS


@functools.partial(jax.jit, static_argnames=("chunk",))
def _sc_gather(gidx, tab_flat, chunk=3328):
    n, = gidx.shape
    _, d = tab_flat.shape
    n_per_w = n // _NW
    assert n_per_w * _NW == n and n_per_w % chunk == 0
    n_ch = n_per_w // chunk

    mesh = plsc.VectorSubcoreMesh(
        core_axis_name="c", subcore_axis_name="s",
        num_cores=_NC, num_subcores=_NS)

    @functools.partial(
        pl.kernel,
        out_type=jax.ShapeDtypeStruct((n, d), jnp.float32),
        mesh=mesh,
        scratch_types=[
            pltpu.VMEM((chunk,), jnp.int32),
            pltpu.VMEM((chunk, d), jnp.float32),
            pltpu.SemaphoreType.DMA,
        ],
        compiler_params=pltpu.CompilerParams(use_tc_tiling_on_sc=False),
    )
    def gather(idx_hbm, tab_hbm, out_hbm, idx_v, rows_v, sem):
        wid = lax.axis_index("s") * _NC + lax.axis_index("c")
        base = wid * n_per_w
        for k in range(n_ch):
            off = base + k * chunk
            pltpu.sync_copy(idx_hbm.at[pl.ds(off, chunk)], idx_v)
            pltpu.async_copy(tab_hbm.at[idx_v], rows_v, sem).wait()
            pltpu.sync_copy(rows_v, out_hbm.at[pl.ds(off, chunk)])

    return gather(gidx, tab_flat)


def _tr_body(t_ref, o_ref):
    x = t_ref[0]
    xt = jnp.transpose(x, (1, 0))
    o_ref[0] = jnp.concatenate(
        [xt[0:3200], xt[3200:6400], xt[6400:9600], xt[9600:12800]], axis=1)


def _mm_body(x_ref, w_ref, b_ref, h_ref):
    h_ref[...] = lax.dot_general(
        x_ref[...], w_ref[...], (((1,), (1,)), ((), ())),
        preferred_element_type=jnp.float32) + b_ref[...]


def _bn_body(h_ref, g_ref, bt_ref, o_ref):
    h = h_ref[...]
    mu = jnp.mean(h, axis=0, keepdims=True)
    var = jnp.mean((h - mu) ** 2, axis=0, keepdims=True)
    o_ref[...] = jnp.maximum(
        (h - mu) * lax.rsqrt(var + _EPS) * g_ref[...] + bt_ref[...], 0.0)


def kernel(features, tables, W, b, gamma, beta):
    bsz, f_num = features.shape
    _, vocab, d = tables.shape
    n = bsz * f_num

    v = features.astype(jnp.int32)
    t, r = v // 12800, v % 12800
    perm = (3200 * t + r % 3200) * 4 + r // 3200
    vocab_pad = 102400
    gidx = (perm
            + (jnp.arange(f_num, dtype=jnp.int32) * vocab_pad)[None, :]).reshape(n)

    tab_t = jnp.transpose(tables, (0, 2, 1))
    tab_p = pl.pallas_call(
        _tr_body,
        grid=(f_num, 8),
        in_specs=[pl.BlockSpec((1, d, 12800), lambda f, t: (f, 0, t))],
        out_specs=pl.BlockSpec((1, 3200, 4 * d), lambda f, t: (f, t, 0)),
        out_shape=jax.ShapeDtypeStruct((f_num, 25600, 4 * d), jnp.float32),
    )(tab_t)
    tab_flat = tab_p.reshape(f_num * 102400, d)

    x = _sc_gather(gidx, tab_flat).reshape(bsz, f_num * d)

    blk = 512
    h = pl.pallas_call(
        _mm_body,
        grid=(bsz // blk,),
        in_specs=[
            pl.BlockSpec((blk, f_num * d), lambda i: (i, 0)),
            pl.BlockSpec((d, f_num * d), lambda i: (0, 0)),
            pl.BlockSpec((1, d), lambda i: (0, 0)),
        ],
        out_specs=pl.BlockSpec((blk, d), lambda i: (i, 0)),
        out_shape=jax.ShapeDtypeStruct((bsz, d), jnp.float32),
    )(x, W, b.reshape(1, d))

    out = pl.pallas_call(
        _bn_body,
        out_shape=jax.ShapeDtypeStruct((bsz, d), jnp.float32),
    )(h, gamma.reshape(1, d), beta.reshape(1, d))
    return out


# speedup vs baseline: 26.3448x; 2.0414x over previous
import functools

import jax
import jax.numpy as jnp
from jax import lax
from jax.experimental import pallas as pl
from jax.experimental.pallas import tpu as pltpu
from jax.experimental.pallas import tpu_sc as plsc

_EPS = 1e-5

_NC = 2
_NS = 16
_NW = _NC * _NS


@functools.partial(jax.jit, static_argnames=("chunk",))
def _sc_gather(gidx, tab_flat, chunk=3328):
    n, = gidx.shape
    _, d = tab_flat.shape
    n_per_w = n // _NW
    assert n_per_w * _NW == n and n_per_w % chunk == 0
    n_ch = n_per_w // chunk

    mesh = plsc.VectorSubcoreMesh(
        core_axis_name="c", subcore_axis_name="s",
        num_cores=_NC, num_subcores=_NS)

    @functools.partial(
        pl.kernel,
        out_type=jax.ShapeDtypeStruct((n, d), jnp.float32),
        mesh=mesh,
        scratch_types=[
            pltpu.VMEM((chunk,), jnp.int32),
            pltpu.VMEM((chunk, d), jnp.float32),
            pltpu.SemaphoreType.DMA,
        ],
        compiler_params=pltpu.CompilerParams(use_tc_tiling_on_sc=False),
    )
    def gather(idx_hbm, tab_hbm, out_hbm, idx_v, rows_v, sem):
        wid = lax.axis_index("s") * _NC + lax.axis_index("c")
        base = wid * n_per_w
        for k in range(n_ch):
            off = base + k * chunk
            pltpu.sync_copy(idx_hbm.at[pl.ds(off, chunk)], idx_v)
            pltpu.async_copy(tab_hbm.at[idx_v], rows_v, sem).wait()
            pltpu.sync_copy(rows_v, out_hbm.at[pl.ds(off, chunk)])

    return gather(gidx, tab_flat)


def _tr_body(t_ref, o_ref):
    o_ref[0] = jnp.transpose(t_ref[...], (1, 0))


def _mm_body(x_ref, w_ref, b_ref, h_ref):
    h_ref[...] = lax.dot_general(
        x_ref[...], w_ref[...], (((1,), (1,)), ((), ())),
        preferred_element_type=jnp.float32) + b_ref[...]


def _bn_body(h_ref, g_ref, bt_ref, o_ref):
    h = h_ref[...]
    mu = jnp.mean(h, axis=0, keepdims=True)
    var = jnp.mean((h - mu) ** 2, axis=0, keepdims=True)
    o_ref[...] = jnp.maximum(
        (h - mu) * lax.rsqrt(var + _EPS) * g_ref[...] + bt_ref[...], 0.0)


def kernel(features, tables, W, b, gamma, beta):
    bsz, f_num = features.shape
    _, vocab, d = tables.shape
    n = bsz * f_num

    v = features.astype(jnp.int32)
    farange = jnp.arange(f_num, dtype=jnp.int32)
    gidx = ((farange // 4 * vocab)[None, :] + v) * 4 + (farange % 4)[None, :]
    gidx = gidx.reshape(n)

    n_grp = (f_num + 3) // 4
    tab_t = jnp.transpose(tables, (0, 2, 1)).reshape(f_num * d, vocab)
    tab_p = pl.pallas_call(
        _tr_body,
        grid=(n_grp, 8),
        in_specs=[pl.BlockSpec((4 * d, 12800), lambda g, t: (g, t))],
        out_specs=pl.BlockSpec((1, 12800, 4 * d), lambda g, t: (g, t, 0)),
        out_shape=jax.ShapeDtypeStruct((n_grp, vocab, 4 * d), jnp.float32),
    )(tab_t)
    tab_flat = tab_p.reshape(n_grp * vocab * 4, d)

    x = _sc_gather(gidx, tab_flat).reshape(bsz, f_num * d)

    blk = 512
    h = pl.pallas_call(
        _mm_body,
        grid=(bsz // blk,),
        in_specs=[
            pl.BlockSpec((blk, f_num * d), lambda i: (i, 0)),
            pl.BlockSpec((d, f_num * d), lambda i: (0, 0)),
            pl.BlockSpec((1, d), lambda i: (0, 0)),
        ],
        out_specs=pl.BlockSpec((blk, d), lambda i: (i, 0)),
        out_shape=jax.ShapeDtypeStruct((bsz, d), jnp.float32),
    )(x, W, b.reshape(1, d))

    out = pl.pallas_call(
        _bn_body,
        out_shape=jax.ShapeDtypeStruct((bsz, d), jnp.float32),
    )(h, gamma.reshape(1, d), beta.reshape(1, d))
    return out


# speedup vs baseline: 26.6428x; 1.0113x over previous
import functools

import jax
import jax.numpy as jnp
from jax import lax
from jax.experimental import pallas as pl
from jax.experimental.pallas import tpu as pltpu
from jax.experimental.pallas import tpu_sc as plsc

_EPS = 1e-5

_NC = 2
_NS = 16
_NW = _NC * _NS


@functools.partial(jax.jit, static_argnames=("chunk",))
def _sc_gather(gidx, tab_flat, chunk=3328):
    n, = gidx.shape
    _, d = tab_flat.shape
    n_per_w = n // _NW
    assert n_per_w * _NW == n and n_per_w % chunk == 0
    n_ch = n_per_w // chunk

    mesh = plsc.VectorSubcoreMesh(
        core_axis_name="c", subcore_axis_name="s",
        num_cores=_NC, num_subcores=_NS)

    @functools.partial(
        pl.kernel,
        out_type=jax.ShapeDtypeStruct((n, d), jnp.float32),
        mesh=mesh,
        scratch_types=[
            pltpu.VMEM((chunk,), jnp.int32),
            pltpu.VMEM((chunk, d), jnp.float32),
            pltpu.SemaphoreType.DMA,
        ],
        compiler_params=pltpu.CompilerParams(use_tc_tiling_on_sc=False),
    )
    def gather(idx_hbm, tab_hbm, out_hbm, idx_v, rows_v, sem):
        wid = lax.axis_index("s") * _NC + lax.axis_index("c")
        base = wid * n_per_w
        for k in range(n_ch):
            off = base + k * chunk
            pltpu.sync_copy(idx_hbm.at[pl.ds(off, chunk)], idx_v)
            pltpu.async_copy(tab_hbm.at[idx_v], rows_v, sem).wait()
            pltpu.sync_copy(rows_v, out_hbm.at[pl.ds(off, chunk)])

    return gather(gidx, tab_flat)


def _tr_body(t_ref, o_ref):
    o_ref[0] = jnp.transpose(t_ref[...], (1, 0))


def _mm2_body(x_ref, w_ref, b_ref, h_ref):
    s = pl.program_id(1)

    @pl.when(s == 0)
    def _():
        h_ref[...] = jnp.broadcast_to(b_ref[0], h_ref.shape)

    h_ref[...] += lax.dot_general(
        x_ref[0], w_ref[0], (((1,), (0,)), ((), ())),
        preferred_element_type=jnp.float32)


def _bn_body(h_ref, g_ref, bt_ref, o_ref):
    h = h_ref[...]
    dout2 = h.shape[1]
    s = jnp.sum(h, axis=0, keepdims=True)
    s2 = jnp.sum(h * h, axis=0, keepdims=True)
    cnt = 2.0 * h.shape[0]
    mu = (s[:, :dout2 // 2] + s[:, dout2 // 2:]) / cnt
    ex2 = (s2[:, :dout2 // 2] + s2[:, dout2 // 2:]) / cnt
    var = ex2 - mu * mu
    mu2 = jnp.concatenate([mu, mu], axis=1)
    rs2 = jnp.concatenate([lax.rsqrt(var + _EPS)] * 2, axis=1)
    o_ref[...] = jnp.maximum((h - mu2) * rs2 * g_ref[...] + bt_ref[...], 0.0)


def kernel(features, tables, W, b, gamma, beta):
    bsz, f_num = features.shape
    _, vocab, d = tables.shape
    n = bsz * f_num

    v = features.astype(jnp.int32)
    farange = jnp.arange(f_num, dtype=jnp.int32)
    gidx = ((farange // 4 * vocab)[None, :] + v) * 4 + (farange % 4)[None, :]
    gidx = gidx.reshape(n)

    n_grp = (f_num + 3) // 4
    tab_t = jnp.transpose(tables, (0, 2, 1)).reshape(f_num * d, vocab)
    tab_p = pl.pallas_call(
        _tr_body,
        grid=(n_grp, 8),
        in_specs=[pl.BlockSpec((4 * d, 12800), lambda g, t: (g, t))],
        out_specs=pl.BlockSpec((1, 12800, 4 * d), lambda g, t: (g, t, 0)),
        out_shape=jax.ShapeDtypeStruct((n_grp, vocab, 4 * d), jnp.float32),
    )(tab_t)
    tab_flat = tab_p.reshape(n_grp * vocab * 4, d)

    npair = bsz // 2
    p_iota = jnp.arange(n, dtype=jnp.int32)
    s_p, rem = p_iota // (4 * npair), p_iota % (4 * npair)
    perm = 52 * (rem // 4) + 4 * s_p + rem % 4
    gidx_p = jnp.take(gidx, perm)
    x3 = _sc_gather(gidx_p, tab_flat).reshape(13, npair, 4 * d)

    fan_in = f_num * d
    z = jnp.zeros((fan_in, d), jnp.float32)
    wc3 = jnp.concatenate(
        [jnp.concatenate([W.T, z], axis=1),
         jnp.concatenate([z, W.T], axis=1)], axis=0).reshape(13, 4 * d, 2 * d)
    bc = jnp.concatenate([b, b]).reshape(1, 2 * d)

    blk = 2048
    h2 = pl.pallas_call(
        _mm2_body,
        grid=(npair // blk, 13),
        in_specs=[
            pl.BlockSpec((1, blk, 4 * d), lambda i, s: (s, i, 0)),
            pl.BlockSpec((1, 4 * d, 2 * d), lambda i, s: (s, 0, 0)),
            pl.BlockSpec((1, 2 * d), lambda i, s: (0, 0)),
        ],
        out_specs=pl.BlockSpec((blk, 2 * d), lambda i, s: (i, 0)),
        out_shape=jax.ShapeDtypeStruct((npair, 2 * d), jnp.float32),
    )(x3, wc3, bc)

    g2 = jnp.concatenate([gamma, gamma]).reshape(1, 2 * d)
    bt2 = jnp.concatenate([beta, beta]).reshape(1, 2 * d)
    out2 = pl.pallas_call(
        _bn_body,
        out_shape=jax.ShapeDtypeStruct((bsz // 2, 2 * d), jnp.float32),
    )(h2, g2, bt2)
    return out2.reshape(bsz, d)
